# baseline (speedup 1.0000x reference)
import functools

import jax
import jax.numpy as jnp
from jax import lax
from jax.experimental import pallas as pl
from jax.experimental.pallas import tpu as pltpu
from jax.experimental.pallas import tpu_sc as plsc

_BN_K = 0.9999950000374997
_CH = 128


def _mm_body(x_ref, w_ref, o_ref):
  o_ref[...] = jnp.dot(x_ref[...], w_ref[...],
                       preferred_element_type=jnp.float32)


def _pre_body(degp_ref, y_ref, hp_ref, dinv_ref):
  deg = degp_ref[0] + degp_ref[1] - 1.0
  dinv = lax.rsqrt(deg)
  dinv_ref[...] = dinv
  hp_ref[...] = y_ref[...] * dinv


def _mid_body(pp_ref, hp_ref, dinv_ref, w_ref, g_ref, b_ref, be_ref, o_ref):
  dinv = dinv_ref[...]
  agg = (pp_ref[0] + pp_ref[1] - hp_ref[...]) * dinv
  scale = g_ref[...] * _BN_K
  h = jnp.maximum(agg * scale + (b_ref[...] * scale + be_ref[...]), 0.0)
  o_ref[...] = jnp.dot(h, w_ref[...],
                       preferred_element_type=jnp.float32) * dinv


def _final_body(pp_ref, hp_ref, dinv_ref, dep_ref, g_ref, b_ref, be_ref,
                wh1_ref, bh1_ref, wh2_ref, bh2_ref, o_ref, acc_ref, *,
                rb, nblk):
  k = pl.program_id(0)

  @pl.when(k == 0)
  def _():
    acc_ref[...] = jnp.zeros_like(acc_ref)

  agg = (pp_ref[0] + pp_ref[1] - hp_ref[...]) * dinv_ref[...]
  scale = g_ref[...] * _BN_K
  h = jnp.maximum(agg * scale + (b_ref[...] * scale + be_ref[...]), 0.0)
  ids = lax.broadcasted_iota(jnp.int32, (dep_ref.shape[0], rb), 1) + k * rb
  oh = (dep_ref[...] == ids).astype(jnp.float32)
  acc_ref[...] += jnp.dot(oh, h, preferred_element_type=jnp.float32)

  @pl.when(k == nblk - 1)
  def _():
    t = jnp.maximum(
        jnp.dot(acc_ref[...], wh1_ref[...],
                preferred_element_type=jnp.float32) + bh1_ref[...], 0.0)
    o_ref[...] = jnp.dot(t, wh2_ref[...],
                         preferred_element_type=jnp.float32) + bh2_ref[...]


def _sc_mesh():
  return plsc.VectorSubcoreMesh(core_axis_name="c", subcore_axis_name="s")


def _make_deg(npad, gch, ns, nc):
  chunk = npad // ns

  @functools.partial(
      pl.kernel,
      out_type=jax.ShapeDtypeStruct((nc, npad), jnp.float32),
      mesh=_sc_mesh(),
      scratch_types=[
          pltpu.VMEM((chunk,), jnp.float32),
          pltpu.VMEM((_CH,), jnp.float32),
          pltpu.VMEM((gch, _CH), jnp.int32),
          pltpu.VMEM_SHARED((npad,), jnp.float32),
      ],
      compiler_params=pltpu.CompilerParams(use_tc_tiling_on_sc=False),
  )
  def deg(dst_hbm, out_hbm, fill_v, ones_v, idxd_v, accum):
    cid = lax.axis_index("c")
    sid = lax.axis_index("s")
    wid = sid * nc + cid
    for i in range(chunk // 16):
      fill_v[pl.ds(i * 16, 16)] = jnp.ones((16,), jnp.float32)
    for i in range(_CH // 16):
      ones_v[pl.ds(i * 16, 16)] = jnp.ones((16,), jnp.float32)
    pltpu.sync_copy(fill_v, accum.at[pl.ds(sid * chunk, chunk)])
    pltpu.sync_copy(dst_hbm.at[wid], idxd_v)
    plsc.subcore_barrier()

    def body(gi, c):
      pltpu.sync_copy(ones_v, accum.at[idxd_v.at[gi]], add=True)
      return c

    lax.fori_loop(0, gch, body, 0)
    plsc.subcore_barrier()
    pltpu.sync_copy(accum.at[pl.ds(sid * chunk, chunk)],
                    out_hbm.at[cid, pl.ds(sid * chunk, chunk)])

  return deg


_NB = 8


def _make_agg(npad, hdim, gch, ns, nc):
  chunk = npad // ns

  @functools.partial(
      pl.kernel,
      out_type=jax.ShapeDtypeStruct((nc, npad, hdim), jnp.float32),
      mesh=_sc_mesh(),
      scratch_types=[
          pltpu.VMEM((gch, _CH), jnp.int32),
          pltpu.VMEM((gch, _CH), jnp.int32),
          pltpu.VMEM((_NB, _CH, hdim), jnp.float32),
          pltpu.SemaphoreType.DMA((_NB,)),
          pltpu.SemaphoreType.DMA((_NB,)),
          pltpu.VMEM_SHARED((npad, hdim), jnp.float32),
      ],
      compiler_params=pltpu.CompilerParams(use_tc_tiling_on_sc=False),
  )
  def agg(src_hbm, dst_hbm, hp_hbm, out_hbm, idxs_v, idxd_v, rows_v, gsem,
          ssem, accum):
    cid = lax.axis_index("c")
    sid = lax.axis_index("s")
    wid = sid * nc + cid
    pltpu.sync_copy(hp_hbm.at[pl.ds(sid * chunk, chunk)],
                    accum.at[pl.ds(sid * chunk, chunk)])
    pltpu.sync_copy(src_hbm.at[wid], idxs_v)
    pltpu.sync_copy(dst_hbm.at[wid], idxd_v)
    plsc.subcore_barrier()

    def body(go, c):
      base = go * _NB
      gd = []
      for b in range(_NB):
        gd.append(pltpu.async_copy(hp_hbm.at[idxs_v.at[base + b]],
                                   rows_v.at[b], gsem.at[b]))
      sd = []
      for b in range(_NB):
        gd[b].wait()
        sd.append(pltpu.async_copy(rows_v.at[b], accum.at[idxd_v.at[base + b]],
                                   ssem.at[b], add=True))
      for b in range(_NB):
        sd[b].wait()
      return c

    lax.fori_loop(0, gch // _NB, body, 0)
    plsc.subcore_barrier()
    pltpu.sync_copy(accum.at[pl.ds(sid * chunk, chunk)],
                    out_hbm.at[cid, pl.ds(sid * chunk, chunk)])

  return agg


def kernel(x, edge_index, batch, deployer, W1, b1, g1, be1, W2, b2, g2, be2,
           W3, b3, g3, be3, Wh1, bh1, Wh2, bh2):
  n, in_dim = x.shape
  e = edge_index.shape[1]
  hdim = W1.shape[1]
  nq = deployer.shape[0]

  info = plsc.get_sparse_core_info()
  nc, ns = info.num_cores, info.num_subcores
  nw = nc * ns

  npad = -(-(n + 1) // (16 * ns)) * (16 * ns)
  gch = -(-e // (nw * _CH * _NB)) * _NB
  ep = nw * gch * _CH

  pad = jnp.full((ep - e,), n, jnp.int32)
  srcp = jnp.concatenate([edge_index[0], pad]).reshape(nw, gch, _CH)
  dstp = jnp.concatenate([edge_index[1], pad]).reshape(nw, gch, _CH)
  xp = jnp.pad(x, ((0, npad - n), (0, 0)))

  deg_fn = _make_deg(npad, gch, ns, nc)
  agg_fn = _make_agg(npad, hdim, gch, ns, nc)

  degp = deg_fn(dstp)

  blk = npad // 8
  y1 = pl.pallas_call(
      _mm_body,
      grid=(8,),
      in_specs=[pl.BlockSpec((blk, in_dim), lambda i: (i, 0)),
                pl.BlockSpec((in_dim, hdim), lambda i: (0, 0))],
      out_specs=pl.BlockSpec((blk, hdim), lambda i: (i, 0)),
      out_shape=jax.ShapeDtypeStruct((npad, hdim), jnp.float32),
  )(xp, W1)

  hp1, dinv = pl.pallas_call(
      _pre_body,
      out_shape=(jax.ShapeDtypeStruct((npad, hdim), jnp.float32),
                 jax.ShapeDtypeStruct((npad, 1), jnp.float32)),
  )(degp.reshape(nc, npad, 1), y1)

  def mid(pp, hp, w, g, b, be):
    return pl.pallas_call(
        _mid_body,
        out_shape=jax.ShapeDtypeStruct((npad, hdim), jnp.float32),
    )(pp, hp, dinv, w, g.reshape(1, hdim), b.reshape(1, hdim),
      be.reshape(1, hdim))

  pp1 = agg_fn(srcp, dstp, hp1)
  hp2 = mid(pp1, hp1, W2, g1, b1, be1)
  pp2 = agg_fn(srcp, dstp, hp2)
  hp3 = mid(pp2, hp2, W3, g2, b2, be2)
  pp3 = agg_fn(srcp, dstp, hp3)

  nblk = 8
  rb = npad // nblk
  out = pl.pallas_call(
      functools.partial(_final_body, rb=rb, nblk=nblk),
      grid=(nblk,),
      in_specs=[
          pl.BlockSpec((nc, rb, hdim), lambda k: (0, k, 0)),
          pl.BlockSpec((rb, hdim), lambda k: (k, 0)),
          pl.BlockSpec((rb, 1), lambda k: (k, 0)),
          pl.BlockSpec((nq, 1), lambda k: (0, 0)),
          pl.BlockSpec((1, hdim), lambda k: (0, 0)),
          pl.BlockSpec((1, hdim), lambda k: (0, 0)),
          pl.BlockSpec((1, hdim), lambda k: (0, 0)),
          pl.BlockSpec(Wh1.shape, lambda k: (0, 0)),
          pl.BlockSpec((1, Wh1.shape[1]), lambda k: (0, 0)),
          pl.BlockSpec(Wh2.shape, lambda k: (0, 0)),
          pl.BlockSpec((1, 1), lambda k: (0, 0)),
      ],
      out_specs=pl.BlockSpec((nq, 1), lambda k: (0, 0)),
      out_shape=jax.ShapeDtypeStruct((nq, 1), jnp.float32),
      scratch_shapes=[pltpu.VMEM((nq, hdim), jnp.float32)],
  )(pp3, hp3, dinv, deployer.reshape(nq, 1), g3.reshape(1, hdim),
    b3.reshape(1, hdim), be3.reshape(1, hdim),
    Wh1, bh1.reshape(1, -1), Wh2, bh2.reshape(1, 1))

  return out.reshape(nq)


# speedup vs baseline: 1.8296x; 1.8296x over previous
import functools

import jax
import jax.numpy as jnp
from jax import lax
from jax.experimental import pallas as pl
from jax.experimental.pallas import tpu as pltpu
from jax.experimental.pallas import tpu_sc as plsc

_BN_K = 0.9999950000374997
_CH = 128
_NB = 8


def _mm_body(x_ref, w_ref, o_ref):
  o_ref[...] = jnp.dot(x_ref[...], w_ref[...],
                       preferred_element_type=jnp.float32)


def _split(y, o_ref, hh):
  o_ref[0] = y[:, :hh]
  o_ref[1] = y[:, hh:]


def _pre_body(degp_ref, y_ref, hps_ref, dinv_ref):
  deg = degp_ref[0] + degp_ref[1] - 1.0
  dinv = lax.rsqrt(deg)
  dinv_ref[...] = dinv
  _split(y_ref[...] * dinv, hps_ref, hps_ref.shape[2])


def _layer(pp_ref, dinv_ref, g_ref, b_ref, be_ref):
  agg = pp_ref[...] * dinv_ref[...]
  scale = g_ref[...] * _BN_K
  h = jnp.maximum(agg * scale + (b_ref[...] * scale + be_ref[...]), 0.0)
  return jnp.concatenate([h[0], h[1]], axis=1)


def _mid_body(pp_ref, dinv_ref, w_ref, g_ref, b_ref, be_ref, o_ref):
  h = _layer(pp_ref, dinv_ref, g_ref, b_ref, be_ref)
  y = jnp.dot(h, w_ref[...], preferred_element_type=jnp.float32)
  _split(y * dinv_ref[...], o_ref, o_ref.shape[2])


def _final_body(pp_ref, dinv_ref, dep_ref, g_ref, b_ref, be_ref,
                wh1_ref, bh1_ref, wh2_ref, bh2_ref, o_ref, acc_ref, *,
                rb, nblk):
  k = pl.program_id(0)

  @pl.when(k == 0)
  def _():
    acc_ref[...] = jnp.zeros_like(acc_ref)

  h = _layer(pp_ref, dinv_ref, g_ref, b_ref, be_ref)
  ids = lax.broadcasted_iota(jnp.int32, (dep_ref.shape[0], rb), 1) + k * rb
  oh = (dep_ref[...] == ids).astype(jnp.float32)
  acc_ref[...] += jnp.dot(oh, h, preferred_element_type=jnp.float32)

  @pl.when(k == nblk - 1)
  def _():
    t = jnp.maximum(
        jnp.dot(acc_ref[...], wh1_ref[...],
                preferred_element_type=jnp.float32) + bh1_ref[...], 0.0)
    o_ref[...] = jnp.dot(t, wh2_ref[...],
                         preferred_element_type=jnp.float32) + bh2_ref[...]


def _sc_mesh():
  return plsc.VectorSubcoreMesh(core_axis_name="c", subcore_axis_name="s")


def _make_deg(npad, gct, ns, nc):
  chunk = npad // ns
  gh = gct // nc

  @functools.partial(
      pl.kernel,
      out_type=jax.ShapeDtypeStruct((nc, npad), jnp.float32),
      mesh=_sc_mesh(),
      scratch_types=[
          pltpu.VMEM((chunk,), jnp.float32),
          pltpu.VMEM((_CH,), jnp.float32),
          pltpu.VMEM((gct, _CH), jnp.int32),
          pltpu.VMEM_SHARED((npad,), jnp.float32),
      ],
      compiler_params=pltpu.CompilerParams(use_tc_tiling_on_sc=False),
  )
  def deg(dst_hbm, out_hbm, fill_v, ones_v, idxd_v, accum):
    cid = lax.axis_index("c")
    sid = lax.axis_index("s")
    for i in range(chunk // 16):
      fill_v[pl.ds(i * 16, 16)] = jnp.ones((16,), jnp.float32)
    for i in range(_CH // 16):
      ones_v[pl.ds(i * 16, 16)] = jnp.ones((16,), jnp.float32)
    pltpu.sync_copy(fill_v, accum.at[pl.ds(sid * chunk, chunk)])
    pltpu.sync_copy(dst_hbm.at[sid], idxd_v)
    plsc.subcore_barrier()

    def body(gi, c):
      pltpu.sync_copy(ones_v, accum.at[idxd_v.at[cid * gh + gi]], add=True)
      return c

    lax.fori_loop(0, gh, body, 0)
    plsc.subcore_barrier()
    pltpu.sync_copy(accum.at[pl.ds(sid * chunk, chunk)],
                    out_hbm.at[cid, pl.ds(sid * chunk, chunk)])

  return deg


def _make_agg(npad, hh, gct, ns, nc):
  chunk = npad // ns

  @functools.partial(
      pl.kernel,
      out_type=jax.ShapeDtypeStruct((nc, npad, hh), jnp.float32),
      mesh=_sc_mesh(),
      scratch_types=[
          pltpu.VMEM((gct, _CH), jnp.int32),
          pltpu.VMEM((gct, _CH), jnp.int32),
          pltpu.VMEM((_NB, _CH, hh), jnp.float32),
          pltpu.SemaphoreType.DMA((_NB,)),
          pltpu.SemaphoreType.DMA((_NB,)),
          pltpu.VMEM_SHARED((npad, hh), jnp.float32),
          pltpu.VMEM_SHARED((npad, hh), jnp.float32),
      ],
      compiler_params=pltpu.CompilerParams(use_tc_tiling_on_sc=False),
  )
  def agg(src_hbm, dst_hbm, hps_hbm, out_hbm, idxs_v, idxd_v, rows_v, gsem,
          ssem, accum, table):
    cid = lax.axis_index("c")
    sid = lax.axis_index("s")
    pltpu.sync_copy(hps_hbm.at[cid, pl.ds(sid * chunk, chunk)],
                    accum.at[pl.ds(sid * chunk, chunk)])
    pltpu.sync_copy(hps_hbm.at[cid, pl.ds(sid * chunk, chunk)],
                    table.at[pl.ds(sid * chunk, chunk)])
    pltpu.sync_copy(src_hbm.at[sid], idxs_v)
    pltpu.sync_copy(dst_hbm.at[sid], idxd_v)
    plsc.subcore_barrier()

    def body(go, c):
      base = go * _NB
      gd = []
      for b in range(_NB):
        gd.append(pltpu.async_copy(table.at[idxs_v.at[base + b]],
                                   rows_v.at[b], gsem.at[b]))
      sd = []
      for b in range(_NB):
        gd[b].wait()
        sd.append(pltpu.async_copy(rows_v.at[b], accum.at[idxd_v.at[base + b]],
                                   ssem.at[b], add=True))
      for b in range(_NB):
        sd[b].wait()
      return c

    lax.fori_loop(0, gct // _NB, body, 0)
    plsc.subcore_barrier()
    pltpu.sync_copy(accum.at[pl.ds(sid * chunk, chunk)],
                    out_hbm.at[cid, pl.ds(sid * chunk, chunk)])

  return agg


def kernel(x, edge_index, batch, deployer, W1, b1, g1, be1, W2, b2, g2, be2,
           W3, b3, g3, be3, Wh1, bh1, Wh2, bh2):
  n, in_dim = x.shape
  e = edge_index.shape[1]
  hdim = W1.shape[1]
  hh = hdim // 2
  nq = deployer.shape[0]

  info = plsc.get_sparse_core_info()
  nc, ns = info.num_cores, info.num_subcores

  npad = -(-(n + 1) // (16 * ns)) * (16 * ns)
  gct = -(-e // (ns * _CH * _NB * nc)) * _NB * nc
  ep = ns * gct * _CH

  pad = jnp.full((ep - e,), n, jnp.int32)
  srcp = jnp.concatenate([edge_index[0], pad]).reshape(ns, gct, _CH)
  dstp = jnp.concatenate([edge_index[1], pad]).reshape(ns, gct, _CH)
  xp = jnp.pad(x, ((0, npad - n), (0, 0)))

  degp = _make_deg(npad, gct, ns, nc)(dstp)
  agg_fn = _make_agg(npad, hh, gct, ns, nc)

  blk = npad // 8
  y1 = pl.pallas_call(
      _mm_body,
      grid=(8,),
      in_specs=[pl.BlockSpec((blk, in_dim), lambda i: (i, 0)),
                pl.BlockSpec((in_dim, hdim), lambda i: (0, 0))],
      out_specs=pl.BlockSpec((blk, hdim), lambda i: (i, 0)),
      out_shape=jax.ShapeDtypeStruct((npad, hdim), jnp.float32),
  )(xp, W1)

  hps1, dinv = pl.pallas_call(
      _pre_body,
      out_shape=(jax.ShapeDtypeStruct((nc, npad, hh), jnp.float32),
                 jax.ShapeDtypeStruct((npad, 1), jnp.float32)),
  )(degp.reshape(nc, npad, 1), y1)

  def sp(v):
    return v.reshape(nc, 1, hh)

  def mid(pp, w, g, b, be):
    return pl.pallas_call(
        _mid_body,
        out_shape=jax.ShapeDtypeStruct((nc, npad, hh), jnp.float32),
    )(pp, dinv, w, sp(g), sp(b), sp(be))

  pp1 = agg_fn(srcp, dstp, hps1)
  hps2 = mid(pp1, W2, g1, b1, be1)
  pp2 = agg_fn(srcp, dstp, hps2)
  hps3 = mid(pp2, W3, g2, b2, be2)
  pp3 = agg_fn(srcp, dstp, hps3)

  nblk = 8
  rb = npad // nblk
  out = pl.pallas_call(
      functools.partial(_final_body, rb=rb, nblk=nblk),
      grid=(nblk,),
      in_specs=[
          pl.BlockSpec((nc, rb, hh), lambda k: (0, k, 0)),
          pl.BlockSpec((rb, 1), lambda k: (k, 0)),
          pl.BlockSpec((nq, 1), lambda k: (0, 0)),
          pl.BlockSpec((nc, 1, hh), lambda k: (0, 0, 0)),
          pl.BlockSpec((nc, 1, hh), lambda k: (0, 0, 0)),
          pl.BlockSpec((nc, 1, hh), lambda k: (0, 0, 0)),
          pl.BlockSpec(Wh1.shape, lambda k: (0, 0)),
          pl.BlockSpec((1, Wh1.shape[1]), lambda k: (0, 0)),
          pl.BlockSpec(Wh2.shape, lambda k: (0, 0)),
          pl.BlockSpec((1, 1), lambda k: (0, 0)),
      ],
      out_specs=pl.BlockSpec((nq, 1), lambda k: (0, 0)),
      out_shape=jax.ShapeDtypeStruct((nq, 1), jnp.float32),
      scratch_shapes=[pltpu.VMEM((nq, hdim), jnp.float32)],
  )(pp3, dinv, deployer.reshape(nq, 1), sp(g3), sp(b3), sp(be3),
    Wh1, bh1.reshape(1, -1), Wh2, bh2.reshape(1, 1))

  return out.reshape(nq)


# speedup vs baseline: 1.8368x; 1.0040x over previous
import functools

import jax
import jax.numpy as jnp
from jax import lax
from jax.experimental import pallas as pl
from jax.experimental.pallas import tpu as pltpu
from jax.experimental.pallas import tpu_sc as plsc

_BN_K = 0.9999950000374997
_CH = 128
_NB = 8


def _split(y, o_ref, hh):
  o_ref[0] = y[:, :hh]
  o_ref[1] = y[:, hh:]


def _pre_body(degp_ref, x_ref, w_ref, hps_ref, dinv_ref):
  deg = degp_ref[0] + degp_ref[1] - 1.0
  dinv = lax.rsqrt(deg)
  dinv_ref[...] = dinv
  y = jnp.dot(x_ref[...], w_ref[...], preferred_element_type=jnp.float32)
  _split(y * dinv, hps_ref, hps_ref.shape[2])


def _layer(pp_ref, dinv_ref, g_ref, b_ref, be_ref):
  agg = pp_ref[...] * dinv_ref[...]
  scale = g_ref[...] * _BN_K
  h = jnp.maximum(agg * scale + (b_ref[...] * scale + be_ref[...]), 0.0)
  return jnp.concatenate([h[0], h[1]], axis=1)


def _mid_body(pp_ref, dinv_ref, w_ref, g_ref, b_ref, be_ref, o_ref):
  h = _layer(pp_ref, dinv_ref, g_ref, b_ref, be_ref)
  y = jnp.dot(h, w_ref[...], preferred_element_type=jnp.float32)
  _split(y * dinv_ref[...], o_ref, o_ref.shape[2])


def _final_body(pp_ref, dinv_ref, dep_ref, g_ref, b_ref, be_ref,
                wh1_ref, bh1_ref, wh2_ref, bh2_ref, o_ref, acc_ref, *,
                rb, nblk):
  k = pl.program_id(0)

  @pl.when(k == 0)
  def _():
    acc_ref[...] = jnp.zeros_like(acc_ref)

  h = _layer(pp_ref, dinv_ref, g_ref, b_ref, be_ref)
  ids = lax.broadcasted_iota(jnp.int32, (dep_ref.shape[0], rb), 1) + k * rb
  oh = (dep_ref[...] == ids).astype(jnp.float32)
  acc_ref[...] += jnp.dot(oh, h, preferred_element_type=jnp.float32)

  @pl.when(k == nblk - 1)
  def _():
    t = jnp.maximum(
        jnp.dot(acc_ref[...], wh1_ref[...],
                preferred_element_type=jnp.float32) + bh1_ref[...], 0.0)
    o_ref[...] = jnp.dot(t, wh2_ref[...],
                         preferred_element_type=jnp.float32) + bh2_ref[...]


def _sc_mesh():
  return plsc.VectorSubcoreMesh(core_axis_name="c", subcore_axis_name="s")


def _make_deg(npad, gct, ns, nc):
  chunk = npad // ns
  gh = gct // nc

  @functools.partial(
      pl.kernel,
      out_type=jax.ShapeDtypeStruct((nc, npad), jnp.float32),
      mesh=_sc_mesh(),
      scratch_types=[
          pltpu.VMEM((chunk,), jnp.float32),
          pltpu.VMEM((_CH,), jnp.float32),
          pltpu.VMEM((gct, _CH), jnp.int32),
          pltpu.VMEM_SHARED((npad,), jnp.float32),
      ],
      compiler_params=pltpu.CompilerParams(use_tc_tiling_on_sc=False),
  )
  def deg(dst_hbm, out_hbm, fill_v, ones_v, idxd_v, accum):
    cid = lax.axis_index("c")
    sid = lax.axis_index("s")
    for i in range(chunk // 16):
      fill_v[pl.ds(i * 16, 16)] = jnp.ones((16,), jnp.float32)
    for i in range(_CH // 16):
      ones_v[pl.ds(i * 16, 16)] = jnp.ones((16,), jnp.float32)
    pltpu.sync_copy(fill_v, accum.at[pl.ds(sid * chunk, chunk)])
    pltpu.sync_copy(dst_hbm.at[sid], idxd_v)
    plsc.subcore_barrier()

    def body(gi, c):
      pltpu.sync_copy(ones_v, accum.at[idxd_v.at[cid * gh + gi]], add=True)
      return c

    lax.fori_loop(0, gh, body, 0)
    plsc.subcore_barrier()
    pltpu.sync_copy(accum.at[pl.ds(sid * chunk, chunk)],
                    out_hbm.at[cid, pl.ds(sid * chunk, chunk)])

  return deg


def _make_agg(npad, hh, gct, ns, nc):
  chunk = npad // ns

  @functools.partial(
      pl.kernel,
      out_type=jax.ShapeDtypeStruct((nc, npad, hh), jnp.float32),
      mesh=_sc_mesh(),
      scratch_types=[
          pltpu.VMEM((gct, _CH), jnp.int32),
          pltpu.VMEM((gct, _CH), jnp.int32),
          pltpu.VMEM((_NB, _CH, hh), jnp.float32),
          pltpu.SemaphoreType.DMA((_NB,)),
          pltpu.SemaphoreType.DMA((_NB,)),
          pltpu.VMEM_SHARED((npad, hh), jnp.float32),
          pltpu.VMEM_SHARED((npad, hh), jnp.float32),
      ],
      compiler_params=pltpu.CompilerParams(use_tc_tiling_on_sc=False),
  )
  def agg(src_hbm, dst_hbm, hps_hbm, out_hbm, idxs_v, idxd_v, rows_v, gsem,
          ssem, accum, table):
    cid = lax.axis_index("c")
    sid = lax.axis_index("s")
    pltpu.sync_copy(hps_hbm.at[cid, pl.ds(sid * chunk, chunk)],
                    accum.at[pl.ds(sid * chunk, chunk)])
    pltpu.sync_copy(hps_hbm.at[cid, pl.ds(sid * chunk, chunk)],
                    table.at[pl.ds(sid * chunk, chunk)])
    pltpu.sync_copy(src_hbm.at[sid], idxs_v)
    pltpu.sync_copy(dst_hbm.at[sid], idxd_v)
    plsc.subcore_barrier()

    def body(go, c):
      base = go * _NB
      gd = []
      for b in range(_NB):
        gd.append(pltpu.async_copy(table.at[idxs_v.at[base + b]],
                                   rows_v.at[b], gsem.at[b]))
      sd = []
      for b in range(_NB):
        gd[b].wait()
        sd.append(pltpu.async_copy(rows_v.at[b], accum.at[idxd_v.at[base + b]],
                                   ssem.at[b], add=True))
      for b in range(_NB):
        sd[b].wait()
      return c

    lax.fori_loop(0, gct // _NB, body, 0)
    plsc.subcore_barrier()
    pltpu.sync_copy(accum.at[pl.ds(sid * chunk, chunk)],
                    out_hbm.at[cid, pl.ds(sid * chunk, chunk)])

  return agg


def kernel(x, edge_index, batch, deployer, W1, b1, g1, be1, W2, b2, g2, be2,
           W3, b3, g3, be3, Wh1, bh1, Wh2, bh2):
  n, in_dim = x.shape
  e = edge_index.shape[1]
  hdim = W1.shape[1]
  hh = hdim // 2
  nq = deployer.shape[0]

  info = plsc.get_sparse_core_info()
  nc, ns = info.num_cores, info.num_subcores

  npad = -(-(n + 1) // (16 * ns)) * (16 * ns)
  gct = -(-e // (ns * _CH * _NB * nc)) * _NB * nc
  ep = ns * gct * _CH

  pad = jnp.full((ep - e,), n, jnp.int32)
  srcp = jnp.concatenate([edge_index[0], pad]).reshape(ns, gct, _CH)
  dstp = jnp.concatenate([edge_index[1], pad]).reshape(ns, gct, _CH)
  xp = jnp.pad(x, ((0, npad - n), (0, 0)))

  degp = _make_deg(npad, gct, ns, nc)(dstp)
  agg_fn = _make_agg(npad, hh, gct, ns, nc)

  nblk = 8
  rb = npad // nblk
  hps1, dinv = pl.pallas_call(
      _pre_body,
      grid=(nblk,),
      in_specs=[pl.BlockSpec((nc, rb, 1), lambda i: (0, i, 0)),
                pl.BlockSpec((rb, in_dim), lambda i: (i, 0)),
                pl.BlockSpec((in_dim, hdim), lambda i: (0, 0))],
      out_specs=(pl.BlockSpec((nc, rb, hh), lambda i: (0, i, 0)),
                 pl.BlockSpec((rb, 1), lambda i: (i, 0))),
      out_shape=(jax.ShapeDtypeStruct((nc, npad, hh), jnp.float32),
                 jax.ShapeDtypeStruct((npad, 1), jnp.float32)),
  )(degp.reshape(nc, npad, 1), xp, W1)

  def sp(v):
    return v.reshape(nc, 1, hh)

  def mid(pp, w, g, b, be):
    return pl.pallas_call(
        _mid_body,
        grid=(nblk,),
        in_specs=[pl.BlockSpec((nc, rb, hh), lambda i: (0, i, 0)),
                  pl.BlockSpec((rb, 1), lambda i: (i, 0)),
                  pl.BlockSpec((hdim, hdim), lambda i: (0, 0)),
                  pl.BlockSpec((nc, 1, hh), lambda i: (0, 0, 0)),
                  pl.BlockSpec((nc, 1, hh), lambda i: (0, 0, 0)),
                  pl.BlockSpec((nc, 1, hh), lambda i: (0, 0, 0))],
        out_specs=pl.BlockSpec((nc, rb, hh), lambda i: (0, i, 0)),
        out_shape=jax.ShapeDtypeStruct((nc, npad, hh), jnp.float32),
    )(pp, dinv, w, sp(g), sp(b), sp(be))

  pp1 = agg_fn(srcp, dstp, hps1)
  hps2 = mid(pp1, W2, g1, b1, be1)
  pp2 = agg_fn(srcp, dstp, hps2)
  hps3 = mid(pp2, W3, g2, b2, be2)
  pp3 = agg_fn(srcp, dstp, hps3)

  out = pl.pallas_call(
      functools.partial(_final_body, rb=rb, nblk=nblk),
      grid=(nblk,),
      in_specs=[
          pl.BlockSpec((nc, rb, hh), lambda k: (0, k, 0)),
          pl.BlockSpec((rb, 1), lambda k: (k, 0)),
          pl.BlockSpec((nq, 1), lambda k: (0, 0)),
          pl.BlockSpec((nc, 1, hh), lambda k: (0, 0, 0)),
          pl.BlockSpec((nc, 1, hh), lambda k: (0, 0, 0)),
          pl.BlockSpec((nc, 1, hh), lambda k: (0, 0, 0)),
          pl.BlockSpec(Wh1.shape, lambda k: (0, 0)),
          pl.BlockSpec((1, Wh1.shape[1]), lambda k: (0, 0)),
          pl.BlockSpec(Wh2.shape, lambda k: (0, 0)),
          pl.BlockSpec((1, 1), lambda k: (0, 0)),
      ],
      out_specs=pl.BlockSpec((nq, 1), lambda k: (0, 0)),
      out_shape=jax.ShapeDtypeStruct((nq, 1), jnp.float32),
      scratch_shapes=[pltpu.VMEM((nq, hdim), jnp.float32)],
  )(pp3, dinv, deployer.reshape(nq, 1), sp(g3), sp(b3), sp(be3),
    Wh1, bh1.reshape(1, -1), Wh2, bh2.reshape(1, 1))

  return out.reshape(nq)


# speedup vs baseline: 1.9834x; 1.0798x over previous
import functools

import jax
import jax.numpy as jnp
from jax import lax
from jax.experimental import pallas as pl
from jax.experimental.pallas import tpu as pltpu
from jax.experimental.pallas import tpu_sc as plsc

_BN_K = 0.9999950000374997
_CH = 128
_NB = 8


def _split(y, o_ref, hh):
  o_ref[0] = y[:, :hh]
  o_ref[1] = y[:, hh:]


def _pre_body(degp_ref, x_ref, w_ref, hps_ref, dinv_ref):
  deg = degp_ref[0] + degp_ref[1] - 1.0
  dinv = lax.rsqrt(deg)
  dinv_ref[...] = dinv
  y = jnp.dot(x_ref[...], w_ref[...], preferred_element_type=jnp.float32)
  _split(y * dinv, hps_ref, hps_ref.shape[2])


def _layer(pp_ref, dinv_ref, g_ref, b_ref, be_ref):
  agg = pp_ref[...] * dinv_ref[...]
  scale = g_ref[...] * _BN_K
  h = jnp.maximum(agg * scale + (b_ref[...] * scale + be_ref[...]), 0.0)
  return jnp.concatenate([h[0], h[1]], axis=1)


def _mid_body(pp_ref, dinv_ref, w_ref, g_ref, b_ref, be_ref, o_ref):
  h = _layer(pp_ref, dinv_ref, g_ref, b_ref, be_ref)
  y = jnp.dot(h, w_ref[...], preferred_element_type=jnp.float32)
  _split(y * dinv_ref[...], o_ref, o_ref.shape[2])


def _final_body(pp_ref, dinv_ref, dep_ref, g_ref, b_ref, be_ref,
                wh1_ref, bh1_ref, wh2_ref, bh2_ref, o_ref, acc_ref, *,
                rb, nblk):
  k = pl.program_id(0)

  @pl.when(k == 0)
  def _():
    acc_ref[...] = jnp.zeros_like(acc_ref)

  h = _layer(pp_ref, dinv_ref, g_ref, b_ref, be_ref)
  ids = lax.broadcasted_iota(jnp.int32, (dep_ref.shape[0], rb), 1) + k * rb
  oh = (dep_ref[...] == ids).astype(jnp.float32)
  acc_ref[...] += jnp.dot(oh, h, preferred_element_type=jnp.float32)

  @pl.when(k == nblk - 1)
  def _():
    t = jnp.maximum(
        jnp.dot(acc_ref[...], wh1_ref[...],
                preferred_element_type=jnp.float32) + bh1_ref[...], 0.0)
    o_ref[...] = jnp.dot(t, wh2_ref[...],
                         preferred_element_type=jnp.float32) + bh2_ref[...]


def _sc_mesh():
  return plsc.VectorSubcoreMesh(core_axis_name="c", subcore_axis_name="s")


def _make_deg(npad, gct, ns, nc):
  chunk = npad // ns
  gh = gct // nc

  @functools.partial(
      pl.kernel,
      out_type=jax.ShapeDtypeStruct((nc, npad), jnp.float32),
      mesh=_sc_mesh(),
      scratch_types=[
          pltpu.VMEM((chunk,), jnp.float32),
          pltpu.VMEM((_CH,), jnp.float32),
          pltpu.VMEM((gct, _CH), jnp.int32),
          pltpu.VMEM_SHARED((npad,), jnp.float32),
      ],
      compiler_params=pltpu.CompilerParams(use_tc_tiling_on_sc=False),
  )
  def deg(dst_hbm, out_hbm, fill_v, ones_v, idxd_v, accum):
    cid = lax.axis_index("c")
    sid = lax.axis_index("s")
    for i in range(chunk // 16):
      fill_v[pl.ds(i * 16, 16)] = jnp.ones((16,), jnp.float32)
    for i in range(_CH // 16):
      ones_v[pl.ds(i * 16, 16)] = jnp.ones((16,), jnp.float32)
    pltpu.sync_copy(fill_v, accum.at[pl.ds(sid * chunk, chunk)])
    pltpu.sync_copy(dst_hbm.at[sid], idxd_v)
    plsc.subcore_barrier()

    def body(gi, c):
      pltpu.sync_copy(ones_v, accum.at[idxd_v.at[cid * gh + gi]], add=True)
      return c

    lax.fori_loop(0, gh, body, 0)
    plsc.subcore_barrier()
    pltpu.sync_copy(accum.at[pl.ds(sid * chunk, chunk)],
                    out_hbm.at[cid, pl.ds(sid * chunk, chunk)])

  return deg


def _make_agg(npad, hh, gct, ns, nc):
  chunk = npad // ns

  @functools.partial(
      pl.kernel,
      out_type=jax.ShapeDtypeStruct((nc, npad, hh), jnp.float32),
      mesh=_sc_mesh(),
      scratch_types=[
          pltpu.VMEM((gct, _CH), jnp.int32),
          pltpu.VMEM((gct, _CH), jnp.int32),
          pltpu.VMEM((_NB, _CH, hh), jnp.float32),
          pltpu.SemaphoreType.DMA((_NB,)),
          pltpu.SemaphoreType.DMA((_NB,)),
          pltpu.VMEM_SHARED((npad, hh), jnp.float32),
          pltpu.VMEM_SHARED((npad, hh), jnp.float32),
      ],
      compiler_params=pltpu.CompilerParams(use_tc_tiling_on_sc=False),
  )
  def agg(src_hbm, dst_hbm, hps_hbm, out_hbm, idxs_v, idxd_v, rows_v, gsem,
          ssem, accum, table):
    cid = lax.axis_index("c")
    sid = lax.axis_index("s")
    pltpu.sync_copy(hps_hbm.at[cid, pl.ds(sid * chunk, chunk)],
                    accum.at[pl.ds(sid * chunk, chunk)])
    pltpu.sync_copy(hps_hbm.at[cid, pl.ds(sid * chunk, chunk)],
                    table.at[pl.ds(sid * chunk, chunk)])
    pltpu.sync_copy(src_hbm.at[sid], idxs_v)
    pltpu.sync_copy(dst_hbm.at[sid], idxd_v)
    plsc.subcore_barrier()

    def body(go, c):
      base = go * _NB
      gd = []
      for b in range(_NB):
        gd.append(pltpu.async_copy(table.at[idxs_v.at[base + b]],
                                   rows_v.at[b], gsem.at[b]))
      sd = []
      for b in range(_NB):
        gd[b].wait()
        sd.append(pltpu.async_copy(rows_v.at[b], accum.at[idxd_v.at[base + b]],
                                   ssem.at[b], add=True))
      for b in range(_NB):
        sd[b].wait()
      return c

    lax.fori_loop(0, gct // _NB, body, 0)
    plsc.subcore_barrier()
    pltpu.sync_copy(accum.at[pl.ds(sid * chunk, chunk)],
                    out_hbm.at[cid, pl.ds(sid * chunk, chunk)])

  return agg


def kernel(x, edge_index, batch, deployer, W1, b1, g1, be1, W2, b2, g2, be2,
           W3, b3, g3, be3, Wh1, bh1, Wh2, bh2):
  n, in_dim = x.shape
  e = edge_index.shape[1]
  hdim = W1.shape[1]
  hh = hdim // 2
  nq = deployer.shape[0]

  info = plsc.get_sparse_core_info()
  nc, ns = info.num_cores, info.num_subcores

  npad = -(-(n + 1) // (16 * ns)) * (16 * ns)
  gct = -(-e // (ns * _CH * _NB * nc)) * _NB * nc
  ep = ns * gct * _CH

  pad = n + jnp.arange(ep - e, dtype=jnp.int32) % (npad - n)
  srcp = jnp.concatenate([edge_index[0], pad]).reshape(ns, gct, _CH)
  dstp = jnp.concatenate([edge_index[1], pad]).reshape(ns, gct, _CH)
  xp = jnp.pad(x, ((0, npad - n), (0, 0)))

  degp = _make_deg(npad, gct, ns, nc)(dstp)
  agg_fn = _make_agg(npad, hh, gct, ns, nc)

  nblk = 8
  rb = npad // nblk
  hps1, dinv = pl.pallas_call(
      _pre_body,
      grid=(nblk,),
      in_specs=[pl.BlockSpec((nc, rb, 1), lambda i: (0, i, 0)),
                pl.BlockSpec((rb, in_dim), lambda i: (i, 0)),
                pl.BlockSpec((in_dim, hdim), lambda i: (0, 0))],
      out_specs=(pl.BlockSpec((nc, rb, hh), lambda i: (0, i, 0)),
                 pl.BlockSpec((rb, 1), lambda i: (i, 0))),
      out_shape=(jax.ShapeDtypeStruct((nc, npad, hh), jnp.float32),
                 jax.ShapeDtypeStruct((npad, 1), jnp.float32)),
  )(degp.reshape(nc, npad, 1), xp, W1)

  def sp(v):
    return v.reshape(nc, 1, hh)

  def mid(pp, w, g, b, be):
    return pl.pallas_call(
        _mid_body,
        grid=(nblk,),
        in_specs=[pl.BlockSpec((nc, rb, hh), lambda i: (0, i, 0)),
                  pl.BlockSpec((rb, 1), lambda i: (i, 0)),
                  pl.BlockSpec((hdim, hdim), lambda i: (0, 0)),
                  pl.BlockSpec((nc, 1, hh), lambda i: (0, 0, 0)),
                  pl.BlockSpec((nc, 1, hh), lambda i: (0, 0, 0)),
                  pl.BlockSpec((nc, 1, hh), lambda i: (0, 0, 0))],
        out_specs=pl.BlockSpec((nc, rb, hh), lambda i: (0, i, 0)),
        out_shape=jax.ShapeDtypeStruct((nc, npad, hh), jnp.float32),
    )(pp, dinv, w, sp(g), sp(b), sp(be))

  pp1 = agg_fn(srcp, dstp, hps1)
  hps2 = mid(pp1, W2, g1, b1, be1)
  pp2 = agg_fn(srcp, dstp, hps2)
  hps3 = mid(pp2, W3, g2, b2, be2)
  pp3 = agg_fn(srcp, dstp, hps3)

  out = pl.pallas_call(
      functools.partial(_final_body, rb=rb, nblk=nblk),
      grid=(nblk,),
      in_specs=[
          pl.BlockSpec((nc, rb, hh), lambda k: (0, k, 0)),
          pl.BlockSpec((rb, 1), lambda k: (k, 0)),
          pl.BlockSpec((nq, 1), lambda k: (0, 0)),
          pl.BlockSpec((nc, 1, hh), lambda k: (0, 0, 0)),
          pl.BlockSpec((nc, 1, hh), lambda k: (0, 0, 0)),
          pl.BlockSpec((nc, 1, hh), lambda k: (0, 0, 0)),
          pl.BlockSpec(Wh1.shape, lambda k: (0, 0)),
          pl.BlockSpec((1, Wh1.shape[1]), lambda k: (0, 0)),
          pl.BlockSpec(Wh2.shape, lambda k: (0, 0)),
          pl.BlockSpec((1, 1), lambda k: (0, 0)),
      ],
      out_specs=pl.BlockSpec((nq, 1), lambda k: (0, 0)),
      out_shape=jax.ShapeDtypeStruct((nq, 1), jnp.float32),
      scratch_shapes=[pltpu.VMEM((nq, hdim), jnp.float32)],
  )(pp3, dinv, deployer.reshape(nq, 1), sp(g3), sp(b3), sp(be3),
    Wh1, bh1.reshape(1, -1), Wh2, bh2.reshape(1, 1))

  return out.reshape(nq)


# speedup vs baseline: 1.9934x; 1.0050x over previous
import functools

import jax
import jax.numpy as jnp
from jax import lax
from jax.experimental import pallas as pl
from jax.experimental.pallas import tpu as pltpu
from jax.experimental.pallas import tpu_sc as plsc

_BN_K = 0.9999950000374997
_CH = 128
_NB = 10


def _split(y, o_ref, hh):
  o_ref[0] = y[:, :hh]
  o_ref[1] = y[:, hh:]


def _pre_body(degp_ref, x_ref, w_ref, hps_ref, dinv_ref):
  deg = degp_ref[0] + degp_ref[1] - 1.0
  dinv = lax.rsqrt(deg)
  dinv_ref[...] = dinv
  y = jnp.dot(x_ref[...], w_ref[...], preferred_element_type=jnp.float32)
  _split(y * dinv, hps_ref, hps_ref.shape[2])


def _layer(pp_ref, dinv_ref, g_ref, b_ref, be_ref):
  agg = pp_ref[...] * dinv_ref[...]
  scale = g_ref[...] * _BN_K
  h = jnp.maximum(agg * scale + (b_ref[...] * scale + be_ref[...]), 0.0)
  return jnp.concatenate([h[0], h[1]], axis=1)


def _mid_body(pp_ref, dinv_ref, w_ref, g_ref, b_ref, be_ref, o_ref):
  h = _layer(pp_ref, dinv_ref, g_ref, b_ref, be_ref)
  y = jnp.dot(h, w_ref[...], preferred_element_type=jnp.float32)
  _split(y * dinv_ref[...], o_ref, o_ref.shape[2])


def _final_body(pp_ref, dinv_ref, dep_ref, g_ref, b_ref, be_ref,
                wh1_ref, bh1_ref, wh2_ref, bh2_ref, o_ref, acc_ref, *,
                rb, nblk):
  k = pl.program_id(0)

  @pl.when(k == 0)
  def _():
    acc_ref[...] = jnp.zeros_like(acc_ref)

  h = _layer(pp_ref, dinv_ref, g_ref, b_ref, be_ref)
  ids = lax.broadcasted_iota(jnp.int32, (dep_ref.shape[0], rb), 1) + k * rb
  oh = (dep_ref[...] == ids).astype(jnp.float32)
  acc_ref[...] += jnp.dot(oh, h, preferred_element_type=jnp.float32)

  @pl.when(k == nblk - 1)
  def _():
    t = jnp.maximum(
        jnp.dot(acc_ref[...], wh1_ref[...],
                preferred_element_type=jnp.float32) + bh1_ref[...], 0.0)
    o_ref[...] = jnp.dot(t, wh2_ref[...],
                         preferred_element_type=jnp.float32) + bh2_ref[...]


def _sc_mesh():
  return plsc.VectorSubcoreMesh(core_axis_name="c", subcore_axis_name="s")


def _make_deg(npad, gct, ns, nc):
  chunk = npad // ns
  gh = gct // nc

  @functools.partial(
      pl.kernel,
      out_type=jax.ShapeDtypeStruct((nc, npad), jnp.float32),
      mesh=_sc_mesh(),
      scratch_types=[
          pltpu.VMEM((chunk,), jnp.float32),
          pltpu.VMEM((_CH,), jnp.float32),
          pltpu.VMEM((gct, _CH), jnp.int32),
          pltpu.VMEM_SHARED((npad,), jnp.float32),
      ],
      compiler_params=pltpu.CompilerParams(use_tc_tiling_on_sc=False),
  )
  def deg(dst_hbm, out_hbm, fill_v, ones_v, idxd_v, accum):
    cid = lax.axis_index("c")
    sid = lax.axis_index("s")
    for i in range(chunk // 16):
      fill_v[pl.ds(i * 16, 16)] = jnp.ones((16,), jnp.float32)
    for i in range(_CH // 16):
      ones_v[pl.ds(i * 16, 16)] = jnp.ones((16,), jnp.float32)
    pltpu.sync_copy(fill_v, accum.at[pl.ds(sid * chunk, chunk)])
    pltpu.sync_copy(dst_hbm.at[sid], idxd_v)
    plsc.subcore_barrier()

    def body(gi, c):
      pltpu.sync_copy(ones_v, accum.at[idxd_v.at[cid * gh + gi]], add=True)
      return c

    lax.fori_loop(0, gh, body, 0)
    plsc.subcore_barrier()
    pltpu.sync_copy(accum.at[pl.ds(sid * chunk, chunk)],
                    out_hbm.at[cid, pl.ds(sid * chunk, chunk)])

  return deg


def _make_agg(npad, hh, gct, ns, nc):
  chunk = npad // ns

  @functools.partial(
      pl.kernel,
      out_type=jax.ShapeDtypeStruct((nc, npad, hh), jnp.float32),
      mesh=_sc_mesh(),
      scratch_types=[
          pltpu.VMEM((gct, _CH), jnp.int32),
          pltpu.VMEM((gct, _CH), jnp.int32),
          pltpu.VMEM((_NB, _CH, hh), jnp.float32),
          pltpu.SemaphoreType.DMA((_NB,)),
          pltpu.SemaphoreType.DMA((_NB,)),
          pltpu.VMEM_SHARED((npad, hh), jnp.float32),
          pltpu.VMEM_SHARED((npad, hh), jnp.float32),
      ],
      compiler_params=pltpu.CompilerParams(use_tc_tiling_on_sc=False),
  )
  def agg(src_hbm, dst_hbm, hps_hbm, out_hbm, idxs_v, idxd_v, rows_v, gsem,
          ssem, accum, table):
    cid = lax.axis_index("c")
    sid = lax.axis_index("s")
    pltpu.sync_copy(hps_hbm.at[cid, pl.ds(sid * chunk, chunk)],
                    accum.at[pl.ds(sid * chunk, chunk)])
    pltpu.sync_copy(hps_hbm.at[cid, pl.ds(sid * chunk, chunk)],
                    table.at[pl.ds(sid * chunk, chunk)])
    pltpu.sync_copy(src_hbm.at[sid], idxs_v)
    pltpu.sync_copy(dst_hbm.at[sid], idxd_v)
    plsc.subcore_barrier()

    def body(go, c):
      base = go * _NB
      gd = []
      for b in range(_NB):
        gd.append(pltpu.async_copy(table.at[idxs_v.at[base + b]],
                                   rows_v.at[b], gsem.at[b]))
      sd = []
      for b in range(_NB):
        gd[b].wait()
        sd.append(pltpu.async_copy(rows_v.at[b], accum.at[idxd_v.at[base + b]],
                                   ssem.at[b], add=True))
      for b in range(_NB):
        sd[b].wait()
      return c

    lax.fori_loop(0, gct // _NB, body, 0)
    plsc.subcore_barrier()
    pltpu.sync_copy(accum.at[pl.ds(sid * chunk, chunk)],
                    out_hbm.at[cid, pl.ds(sid * chunk, chunk)])

  return agg


def kernel(x, edge_index, batch, deployer, W1, b1, g1, be1, W2, b2, g2, be2,
           W3, b3, g3, be3, Wh1, bh1, Wh2, bh2):
  n, in_dim = x.shape
  e = edge_index.shape[1]
  hdim = W1.shape[1]
  hh = hdim // 2
  nq = deployer.shape[0]

  info = plsc.get_sparse_core_info()
  nc, ns = info.num_cores, info.num_subcores

  npad = -(-(n + 1) // (16 * ns)) * (16 * ns)
  u = _NB if _NB % nc == 0 else _NB * nc
  gct = -(-e // (ns * _CH * u)) * u
  ep = ns * gct * _CH

  pad = n + jnp.arange(ep - e, dtype=jnp.int32) % (npad - n)
  srcp = jnp.concatenate([edge_index[0], pad]).reshape(ns, gct, _CH)
  dstp = jnp.concatenate([edge_index[1], pad]).reshape(ns, gct, _CH)
  xp = jnp.pad(x, ((0, npad - n), (0, 0)))

  degp = _make_deg(npad, gct, ns, nc)(dstp)
  agg_fn = _make_agg(npad, hh, gct, ns, nc)

  nblk = 8
  rb = npad // nblk
  hps1, dinv = pl.pallas_call(
      _pre_body,
      grid=(nblk,),
      in_specs=[pl.BlockSpec((nc, rb, 1), lambda i: (0, i, 0)),
                pl.BlockSpec((rb, in_dim), lambda i: (i, 0)),
                pl.BlockSpec((in_dim, hdim), lambda i: (0, 0))],
      out_specs=(pl.BlockSpec((nc, rb, hh), lambda i: (0, i, 0)),
                 pl.BlockSpec((rb, 1), lambda i: (i, 0))),
      out_shape=(jax.ShapeDtypeStruct((nc, npad, hh), jnp.float32),
                 jax.ShapeDtypeStruct((npad, 1), jnp.float32)),
  )(degp.reshape(nc, npad, 1), xp, W1)

  def sp(v):
    return v.reshape(nc, 1, hh)

  def mid(pp, w, g, b, be):
    return pl.pallas_call(
        _mid_body,
        grid=(nblk,),
        in_specs=[pl.BlockSpec((nc, rb, hh), lambda i: (0, i, 0)),
                  pl.BlockSpec((rb, 1), lambda i: (i, 0)),
                  pl.BlockSpec((hdim, hdim), lambda i: (0, 0)),
                  pl.BlockSpec((nc, 1, hh), lambda i: (0, 0, 0)),
                  pl.BlockSpec((nc, 1, hh), lambda i: (0, 0, 0)),
                  pl.BlockSpec((nc, 1, hh), lambda i: (0, 0, 0))],
        out_specs=pl.BlockSpec((nc, rb, hh), lambda i: (0, i, 0)),
        out_shape=jax.ShapeDtypeStruct((nc, npad, hh), jnp.float32),
    )(pp, dinv, w, sp(g), sp(b), sp(be))

  pp1 = agg_fn(srcp, dstp, hps1)
  hps2 = mid(pp1, W2, g1, b1, be1)
  pp2 = agg_fn(srcp, dstp, hps2)
  hps3 = mid(pp2, W3, g2, b2, be2)
  pp3 = agg_fn(srcp, dstp, hps3)

  out = pl.pallas_call(
      functools.partial(_final_body, rb=rb, nblk=nblk),
      grid=(nblk,),
      in_specs=[
          pl.BlockSpec((nc, rb, hh), lambda k: (0, k, 0)),
          pl.BlockSpec((rb, 1), lambda k: (k, 0)),
          pl.BlockSpec((nq, 1), lambda k: (0, 0)),
          pl.BlockSpec((nc, 1, hh), lambda k: (0, 0, 0)),
          pl.BlockSpec((nc, 1, hh), lambda k: (0, 0, 0)),
          pl.BlockSpec((nc, 1, hh), lambda k: (0, 0, 0)),
          pl.BlockSpec(Wh1.shape, lambda k: (0, 0)),
          pl.BlockSpec((1, Wh1.shape[1]), lambda k: (0, 0)),
          pl.BlockSpec(Wh2.shape, lambda k: (0, 0)),
          pl.BlockSpec((1, 1), lambda k: (0, 0)),
      ],
      out_specs=pl.BlockSpec((nq, 1), lambda k: (0, 0)),
      out_shape=jax.ShapeDtypeStruct((nq, 1), jnp.float32),
      scratch_shapes=[pltpu.VMEM((nq, hdim), jnp.float32)],
  )(pp3, dinv, deployer.reshape(nq, 1), sp(g3), sp(b3), sp(be3),
    Wh1, bh1.reshape(1, -1), Wh2, bh2.reshape(1, 1))

  return out.reshape(nq)


# speedup vs baseline: 2.0502x; 1.0285x over previous
import functools

import jax
import jax.numpy as jnp
from jax import lax
from jax.experimental import pallas as pl
from jax.experimental.pallas import tpu as pltpu
from jax.experimental.pallas import tpu_sc as plsc

_BN_K = 0.9999950000374997
_CH = 128
_NB = 10


def _split(y, o_ref, hh):
  o_ref[0] = y[:, :hh]
  o_ref[1] = y[:, hh:]


def _pre_body(degp_ref, x_ref, w_ref, hps_ref, dinv_ref):
  dt = jnp.transpose(degp_ref[...])
  deg = dt[:, 0:1] + dt[:, 1:2] - 1.0
  dinv = lax.rsqrt(deg)
  dinv_ref[...] = jnp.broadcast_to(dinv, dinv_ref.shape)
  y = jnp.dot(x_ref[...], w_ref[...], preferred_element_type=jnp.float32)
  _split(y * dinv, hps_ref, hps_ref.shape[2])


def _layer(pp_ref, dinv_ref, g_ref, b_ref, be_ref):
  agg = pp_ref[...] * dinv_ref[:, 0:1]
  scale = g_ref[...] * _BN_K
  h = jnp.maximum(agg * scale + (b_ref[...] * scale + be_ref[...]), 0.0)
  return jnp.concatenate([h[0], h[1]], axis=1)


def _mid_body(pp_ref, dinv_ref, w_ref, g_ref, b_ref, be_ref, o_ref):
  h = _layer(pp_ref, dinv_ref, g_ref, b_ref, be_ref)
  y = jnp.dot(h, w_ref[...], preferred_element_type=jnp.float32)
  _split(y * dinv_ref[:, 0:1], o_ref, o_ref.shape[2])


def _final_body(pp_ref, dinv_ref, dep_ref, g_ref, b_ref, be_ref,
                wh1_ref, bh1_ref, wh2_ref, bh2_ref, o_ref, acc_ref, *,
                rb, nblk):
  k = pl.program_id(0)

  @pl.when(k == 0)
  def _():
    acc_ref[...] = jnp.zeros_like(acc_ref)

  h = _layer(pp_ref, dinv_ref, g_ref, b_ref, be_ref)
  ids = lax.broadcasted_iota(jnp.int32, (dep_ref.shape[0], rb), 1) + k * rb
  oh = (dep_ref[...] == ids).astype(jnp.float32)
  acc_ref[...] += jnp.dot(oh, h, preferred_element_type=jnp.float32)

  @pl.when(k == nblk - 1)
  def _():
    t = jnp.maximum(
        jnp.dot(acc_ref[...], wh1_ref[...],
                preferred_element_type=jnp.float32) + bh1_ref[...], 0.0)
    o_ref[...] = jnp.dot(t, wh2_ref[...],
                         preferred_element_type=jnp.float32) + bh2_ref[...]


def _sc_mesh():
  return plsc.VectorSubcoreMesh(core_axis_name="c", subcore_axis_name="s")


def _make_deg(npad, gct, ns, nc):
  chunk = npad // ns
  gh = gct // nc

  @functools.partial(
      pl.kernel,
      out_type=jax.ShapeDtypeStruct((nc, npad), jnp.float32),
      mesh=_sc_mesh(),
      scratch_types=[
          pltpu.VMEM((chunk,), jnp.float32),
          pltpu.VMEM((_CH,), jnp.float32),
          pltpu.VMEM((gct, _CH), jnp.int32),
          pltpu.VMEM_SHARED((npad,), jnp.float32),
      ],
      compiler_params=pltpu.CompilerParams(use_tc_tiling_on_sc=False),
  )
  def deg(dst_hbm, out_hbm, fill_v, ones_v, idxd_v, accum):
    cid = lax.axis_index("c")
    sid = lax.axis_index("s")
    for i in range(chunk // 16):
      fill_v[pl.ds(i * 16, 16)] = jnp.ones((16,), jnp.float32)
    for i in range(_CH // 16):
      ones_v[pl.ds(i * 16, 16)] = jnp.ones((16,), jnp.float32)
    pltpu.sync_copy(fill_v, accum.at[pl.ds(sid * chunk, chunk)])
    pltpu.sync_copy(dst_hbm.at[sid], idxd_v)
    plsc.subcore_barrier()

    def body(gi, c):
      pltpu.sync_copy(ones_v, accum.at[idxd_v.at[cid * gh + gi]], add=True)
      return c

    lax.fori_loop(0, gh, body, 0)
    plsc.subcore_barrier()
    pltpu.sync_copy(accum.at[pl.ds(sid * chunk, chunk)],
                    out_hbm.at[cid, pl.ds(sid * chunk, chunk)])

  return deg


def _make_agg(npad, hh, gct, ns, nc):
  chunk = npad // ns

  @functools.partial(
      pl.kernel,
      out_type=jax.ShapeDtypeStruct((nc, npad, hh), jnp.float32),
      mesh=_sc_mesh(),
      scratch_types=[
          pltpu.VMEM((gct, _CH), jnp.int32),
          pltpu.VMEM((gct, _CH), jnp.int32),
          pltpu.VMEM((_NB, _CH, hh), jnp.float32),
          pltpu.SemaphoreType.DMA((_NB,)),
          pltpu.SemaphoreType.DMA((_NB,)),
          pltpu.VMEM_SHARED((npad, hh), jnp.float32),
          pltpu.VMEM_SHARED((npad, hh), jnp.float32),
      ],
      compiler_params=pltpu.CompilerParams(use_tc_tiling_on_sc=False),
  )
  def agg(src_hbm, dst_hbm, hps_hbm, out_hbm, idxs_v, idxd_v, rows_v, gsem,
          ssem, accum, table):
    cid = lax.axis_index("c")
    sid = lax.axis_index("s")
    pltpu.sync_copy(hps_hbm.at[cid, pl.ds(sid * chunk, chunk)],
                    accum.at[pl.ds(sid * chunk, chunk)])
    pltpu.sync_copy(hps_hbm.at[cid, pl.ds(sid * chunk, chunk)],
                    table.at[pl.ds(sid * chunk, chunk)])
    pltpu.sync_copy(src_hbm.at[sid], idxs_v)
    pltpu.sync_copy(dst_hbm.at[sid], idxd_v)
    plsc.subcore_barrier()

    def body(go, c):
      base = go * _NB
      gd = []
      for b in range(_NB):
        gd.append(pltpu.async_copy(table.at[idxs_v.at[base + b]],
                                   rows_v.at[b], gsem.at[b]))
      sd = []
      for b in range(_NB):
        gd[b].wait()
        sd.append(pltpu.async_copy(rows_v.at[b], accum.at[idxd_v.at[base + b]],
                                   ssem.at[b], add=True))
      for b in range(_NB):
        sd[b].wait()
      return c

    lax.fori_loop(0, gct // _NB, body, 0)
    plsc.subcore_barrier()
    pltpu.sync_copy(accum.at[pl.ds(sid * chunk, chunk)],
                    out_hbm.at[cid, pl.ds(sid * chunk, chunk)])

  return agg


def kernel(x, edge_index, batch, deployer, W1, b1, g1, be1, W2, b2, g2, be2,
           W3, b3, g3, be3, Wh1, bh1, Wh2, bh2):
  n, in_dim = x.shape
  e = edge_index.shape[1]
  hdim = W1.shape[1]
  hh = hdim // 2
  nq = deployer.shape[0]

  info = plsc.get_sparse_core_info()
  nc, ns = info.num_cores, info.num_subcores

  npad = -(-(n + 1) // (16 * ns)) * (16 * ns)
  u = _NB if _NB % nc == 0 else _NB * nc
  gct = -(-e // (ns * _CH * u)) * u
  ep = ns * gct * _CH

  pad = n + jnp.arange(ep - e, dtype=jnp.int32) % (npad - n)
  srcp = jnp.concatenate([edge_index[0], pad]).reshape(ns, gct, _CH)
  dstp = jnp.concatenate([edge_index[1], pad]).reshape(ns, gct, _CH)
  xp = jnp.pad(x, ((0, npad - n), (0, 0)))

  degp = _make_deg(npad, gct, ns, nc)(dstp)
  agg_fn = _make_agg(npad, hh, gct, ns, nc)

  nblk = 8
  rb = npad // nblk
  hps1, dinv = pl.pallas_call(
      _pre_body,
      grid=(nblk,),
      in_specs=[pl.BlockSpec((nc, rb), lambda i: (0, i)),
                pl.BlockSpec((rb, in_dim), lambda i: (i, 0)),
                pl.BlockSpec((in_dim, hdim), lambda i: (0, 0))],
      out_specs=(pl.BlockSpec((nc, rb, hh), lambda i: (0, i, 0)),
                 pl.BlockSpec((rb, 8), lambda i: (i, 0))),
      out_shape=(jax.ShapeDtypeStruct((nc, npad, hh), jnp.float32),
                 jax.ShapeDtypeStruct((npad, 8), jnp.float32)),
  )(degp, xp, W1)

  def sp(v):
    return v.reshape(nc, 1, hh)

  def mid(pp, w, g, b, be):
    return pl.pallas_call(
        _mid_body,
        grid=(nblk,),
        in_specs=[pl.BlockSpec((nc, rb, hh), lambda i: (0, i, 0)),
                  pl.BlockSpec((rb, 8), lambda i: (i, 0)),
                  pl.BlockSpec((hdim, hdim), lambda i: (0, 0)),
                  pl.BlockSpec((nc, 1, hh), lambda i: (0, 0, 0)),
                  pl.BlockSpec((nc, 1, hh), lambda i: (0, 0, 0)),
                  pl.BlockSpec((nc, 1, hh), lambda i: (0, 0, 0))],
        out_specs=pl.BlockSpec((nc, rb, hh), lambda i: (0, i, 0)),
        out_shape=jax.ShapeDtypeStruct((nc, npad, hh), jnp.float32),
    )(pp, dinv, w, sp(g), sp(b), sp(be))

  pp1 = agg_fn(srcp, dstp, hps1)
  hps2 = mid(pp1, W2, g1, b1, be1)
  pp2 = agg_fn(srcp, dstp, hps2)
  hps3 = mid(pp2, W3, g2, b2, be2)
  pp3 = agg_fn(srcp, dstp, hps3)

  out = pl.pallas_call(
      functools.partial(_final_body, rb=rb, nblk=nblk),
      grid=(nblk,),
      in_specs=[
          pl.BlockSpec((nc, rb, hh), lambda k: (0, k, 0)),
          pl.BlockSpec((rb, 8), lambda k: (k, 0)),
          pl.BlockSpec((nq, 1), lambda k: (0, 0)),
          pl.BlockSpec((nc, 1, hh), lambda k: (0, 0, 0)),
          pl.BlockSpec((nc, 1, hh), lambda k: (0, 0, 0)),
          pl.BlockSpec((nc, 1, hh), lambda k: (0, 0, 0)),
          pl.BlockSpec(Wh1.shape, lambda k: (0, 0)),
          pl.BlockSpec((1, Wh1.shape[1]), lambda k: (0, 0)),
          pl.BlockSpec(Wh2.shape, lambda k: (0, 0)),
          pl.BlockSpec((1, 1), lambda k: (0, 0)),
      ],
      out_specs=pl.BlockSpec((nq, 1), lambda k: (0, 0)),
      out_shape=jax.ShapeDtypeStruct((nq, 1), jnp.float32),
      scratch_shapes=[pltpu.VMEM((nq, hdim), jnp.float32)],
  )(pp3, dinv, deployer.reshape(nq, 1), sp(g3), sp(b3), sp(be3),
    Wh1, bh1.reshape(1, -1), Wh2, bh2.reshape(1, 1))

  return out.reshape(nq)
